# baseline (speedup 1.0000x reference)
import functools

import jax
import jax.numpy as jnp
from jax import lax
from jax.experimental import pallas as pl
from jax.experimental.pallas import tpu as pltpu
from jax.experimental.pallas import tpu_sc as plsc

_BN = 512
_BK = 2048


def _stats_body(scale_ref, z_ref, emb_ref, idx_ref, ppl_ref,
                en_full, avg_acc, sem,
                *, n_total, k_total, bn, nb_count):
    nb = pl.program_id(0)
    scale = scale_ref[0, 0]

    @pl.when(nb == 0)
    def _init_once():
        pltpu.make_async_copy(emb_ref, en_full, sem).start()
        avg_acc[...] = jnp.zeros_like(avg_acc)
        pltpu.make_async_copy(emb_ref, en_full, sem).wait()
        e = en_full[...]
        en_full[...] = e / jnp.maximum(
            jnp.sqrt(jnp.sum(e * e, axis=1, keepdims=True)), 1e-12)

    z = z_ref[...]
    zn = z / jnp.maximum(
        jnp.sqrt(jnp.sum(z * z, axis=1, keepdims=True)), 1e-12)
    en = en_full[...]
    logits = scale * lax.dot_general(
        zn, en, (((1,), (1,)), ((), ())),
        preferred_element_type=jnp.float32)
    eexp = jnp.exp(logits - jnp.abs(scale))
    ones_col = jnp.ones((k_total, 1), jnp.float32)
    srow = lax.dot_general(
        eexp, ones_col, (((1,), (0,)), ((), ())),
        preferred_element_type=jnp.float32)

    tmax = jnp.max(logits, axis=1, keepdims=True)
    colidx = lax.broadcasted_iota(jnp.int32, (1, k_total), 1)
    idx_ref[...] = jnp.min(jnp.where(logits == tmax, colidx, k_total),
                           axis=1, keepdims=True)

    recip_row = jnp.transpose(1.0 / srow)
    avg_acc[...] += lax.dot_general(
        recip_row, eexp, (((1,), (0,)), ((), ())),
        preferred_element_type=jnp.float32)

    @pl.when(nb == nb_count - 1)
    def _finish_all():
        avg = avg_acc[...] / n_total
        ent = jnp.sum(avg * jnp.log(avg + 1e-10))
        ppl_ref[...] = jnp.exp(-ent).reshape(1, 1)


def _stats_call(z_e, embeddings, scale2d):
    n, d = z_e.shape
    k = embeddings.shape[0]
    nb_count = n // _BN
    body = functools.partial(
        _stats_body, n_total=n, k_total=k, bn=_BN, nb_count=nb_count)
    return pl.pallas_call(
        body,
        grid=(nb_count,),
        in_specs=[
            pl.BlockSpec((1, 1), lambda i: (0, 0)),
            pl.BlockSpec((_BN, d), lambda i: (i, 0)),
            pl.BlockSpec(memory_space=pl.ANY),
        ],
        out_specs=[
            pl.BlockSpec((_BN, 1), lambda i: (i, 0)),
            pl.BlockSpec((1, 1), lambda i: (0, 0)),
        ],
        out_shape=[
            jax.ShapeDtypeStruct((n, 1), jnp.int32),
            jax.ShapeDtypeStruct((1, 1), jnp.float32),
        ],
        scratch_shapes=[
            pltpu.VMEM((k, d), jnp.float32),
            pltpu.VMEM((1, k), jnp.float32),
            pltpu.SemaphoreType.DMA,
        ],
        compiler_params=pltpu.CompilerParams(
            dimension_semantics=("arbitrary",)),
    )(scale2d, z_e, embeddings)


def _gather_call(table, idx2d, z_e, rw16, n, d):
    info = plsc.get_sparse_core_info()
    nc, ns = info.num_cores, info.num_subcores
    nw = nc * ns
    b_per_w = n // nw
    chunks = b_per_w // 128
    mesh = plsc.VectorSubcoreMesh(core_axis_name="c", subcore_axis_name="s")

    @functools.partial(
        pl.kernel, mesh=mesh,
        out_type=[
            jax.ShapeDtypeStruct((n, d), jnp.float32),
            jax.ShapeDtypeStruct((16,), jnp.float32),
        ],
        compiler_params=pltpu.CompilerParams(use_tc_tiling_on_sc=False),
        scratch_types=[
            pltpu.VMEM((chunks, 128), jnp.int32),
            pltpu.VMEM((b_per_w, d), jnp.float32),
            pltpu.VMEM((b_per_w, d), jnp.float32),
            pltpu.VMEM((16,), jnp.float32),
            pltpu.VMEM((16,), jnp.float32),
            pltpu.SemaphoreType.DMA,
            pltpu.SemaphoreType.DMA,
        ],
    )
    def _gather_kernel(table_hbm, idx_hbm, z_hbm, rw_hbm,
                       out_hbm, alpha_hbm,
                       idx_v, rows_v, z_v, rw_v, av, sem, zsem):
        wid = lax.axis_index("s") * nc + lax.axis_index("c")
        base = wid * b_per_w
        zcopy = pltpu.async_copy(z_hbm.at[pl.ds(base, b_per_w)], z_v, zsem)
        pltpu.sync_copy(rw_hbm, rw_v)
        pltpu.sync_copy(idx_hbm.at[pl.ds(wid * chunks, chunks)], idx_v)
        copies = [
            pltpu.async_copy(table_hbm.at[idx_v.at[j]],
                             rows_v.at[pl.ds(j * 128, 128)], sem)
            for j in range(chunks)
        ]
        for c in copies:
            c.wait()
        zcopy.wait()
        a = 1.0 / (1.0 + jnp.exp(-rw_v[...]))
        one_m_a = 1.0 - a
        av[...] = a

        @pl.when(wid == 0)
        def _():
            pltpu.sync_copy(av, alpha_hbm)

        def _blend_row(r, carry):
            for c4 in range(d // 16):
                g = rows_v[r, pl.ds(c4 * 16, 16)]
                zz = z_v[r, pl.ds(c4 * 16, 16)]
                rows_v[r, pl.ds(c4 * 16, 16)] = a * g + one_m_a * zz
            return carry

        lax.fori_loop(0, b_per_w, _blend_row, 0)
        pltpu.sync_copy(rows_v, out_hbm.at[pl.ds(base, b_per_w)])

    return _gather_kernel(table, idx2d, z_e, rw16)


def kernel(z_e, embeddings, logit_scale, residual_weight):
    n, d = z_e.shape
    scale2d = jnp.reshape(logit_scale, (1, 1)).astype(jnp.float32)
    rw16 = jnp.broadcast_to(
        jnp.reshape(residual_weight, (1,)).astype(jnp.float32), (16,))

    idx_col, ppl = _stats_call(z_e, embeddings, scale2d)
    indices = jnp.reshape(idx_col, (n,))

    z_q, alpha16 = _gather_call(
        embeddings, jnp.reshape(indices, (-1, 128)), z_e, rw16, n, d)

    perplexity = jnp.reshape(ppl, ())
    alpha = jnp.reshape(alpha16[0], ())
    commitment_loss = jnp.zeros((), jnp.float32)
    return (z_q, indices, perplexity, alpha, commitment_loss)


# speedup vs baseline: 1.0043x; 1.0043x over previous
import functools

import jax
import jax.numpy as jnp
from jax import lax
from jax.experimental import pallas as pl
from jax.experimental.pallas import tpu as pltpu
from jax.experimental.pallas import tpu_sc as plsc

_BN = 512
_BK = 2048


def _stats_body(scale_ref, z_ref, emb_ref, idx_ref, ppl_ref,
                en_full, avg_acc, sem,
                *, n_total, k_total, bn, nb_count):
    nb = pl.program_id(0)
    scale = scale_ref[0, 0]

    @pl.when(nb == 0)
    def _init_once():
        pltpu.make_async_copy(emb_ref, en_full, sem).start()
        avg_acc[...] = jnp.zeros_like(avg_acc)
        pltpu.make_async_copy(emb_ref, en_full, sem).wait()
        e = en_full[...]
        en_full[...] = e / jnp.maximum(
            jnp.sqrt(jnp.sum(e * e, axis=1, keepdims=True)), 1e-12)

    z = z_ref[...]
    zn = z / jnp.maximum(
        jnp.sqrt(jnp.sum(z * z, axis=1, keepdims=True)), 1e-12)
    en = en_full[...]
    logits = scale * lax.dot_general(
        zn, en, (((1,), (1,)), ((), ())),
        preferred_element_type=jnp.float32)
    eexp = jnp.exp(logits - jnp.abs(scale))
    ones_col = jnp.ones((k_total, 1), jnp.float32)
    srow = lax.dot_general(
        eexp, ones_col, (((1,), (0,)), ((), ())),
        preferred_element_type=jnp.float32)

    tmax = jnp.max(logits, axis=1, keepdims=True)
    colidx = lax.broadcasted_iota(jnp.int32, (1, k_total), 1)
    idx_ref[...] = jnp.min(jnp.where(logits == tmax, colidx, k_total),
                           axis=1, keepdims=True)

    recip_row = jnp.transpose(1.0 / srow)
    avg_acc[...] += lax.dot_general(
        recip_row, eexp, (((1,), (0,)), ((), ())),
        preferred_element_type=jnp.float32)

    @pl.when(nb == nb_count - 1)
    def _finish_all():
        avg = avg_acc[...] / n_total
        ent = jnp.sum(avg * jnp.log(avg + 1e-10))
        ppl_ref[...] = jnp.exp(-ent).reshape(1, 1)


def _stats_call(z_e, embeddings, scale2d):
    n, d = z_e.shape
    k = embeddings.shape[0]
    nb_count = n // _BN
    body = functools.partial(
        _stats_body, n_total=n, k_total=k, bn=_BN, nb_count=nb_count)
    return pl.pallas_call(
        body,
        grid=(nb_count,),
        in_specs=[
            pl.BlockSpec((1, 1), lambda i: (0, 0)),
            pl.BlockSpec((_BN, d), lambda i: (i, 0)),
            pl.BlockSpec(memory_space=pl.ANY),
        ],
        out_specs=[
            pl.BlockSpec((_BN, 1), lambda i: (i, 0)),
            pl.BlockSpec((1, 1), lambda i: (0, 0)),
        ],
        out_shape=[
            jax.ShapeDtypeStruct((n, 1), jnp.int32),
            jax.ShapeDtypeStruct((1, 1), jnp.float32),
        ],
        scratch_shapes=[
            pltpu.VMEM((k, d), jnp.float32),
            pltpu.VMEM((1, k), jnp.float32),
            pltpu.SemaphoreType.DMA,
        ],
        compiler_params=pltpu.CompilerParams(
            dimension_semantics=("arbitrary",)),
    )(scale2d, z_e, embeddings)


def _gather_call(table, idx2d, n, d):
    info = plsc.get_sparse_core_info()
    nc, ns = info.num_cores, info.num_subcores
    nw = nc * ns
    b_per_w = n // nw
    chunks = b_per_w // 128
    mesh = plsc.VectorSubcoreMesh(core_axis_name="c", subcore_axis_name="s")

    @functools.partial(
        pl.kernel, mesh=mesh,
        out_type=jax.ShapeDtypeStruct((n, d), jnp.float32),
        compiler_params=pltpu.CompilerParams(use_tc_tiling_on_sc=False),
        scratch_types=[
            pltpu.VMEM((chunks, 128), jnp.int32),
            pltpu.VMEM((b_per_w, d), jnp.float32),
            pltpu.SemaphoreType.DMA,
        ],
    )
    def _gather_kernel(table_hbm, idx_hbm, out_hbm, idx_v, rows_v, sem):
        wid = lax.axis_index("s") * nc + lax.axis_index("c")
        pltpu.sync_copy(idx_hbm.at[pl.ds(wid * chunks, chunks)], idx_v)
        copies = [
            pltpu.async_copy(table_hbm.at[idx_v.at[j]],
                             rows_v.at[pl.ds(j * 128, 128)], sem)
            for j in range(chunks)
        ]
        for c in copies:
            c.wait()
        pltpu.sync_copy(rows_v, out_hbm.at[pl.ds(wid * b_per_w, b_per_w)])

    return _gather_kernel(table, idx2d)


def _blend_body(rw_ref, z_ref, g_ref, out_ref, alpha_ref):
    a = 1.0 / (1.0 + jnp.exp(-rw_ref[0, 0]))
    out_ref[...] = a * g_ref[...] + (1.0 - a) * z_ref[...]

    @pl.when(pl.program_id(0) == 0)
    def _():
        alpha_ref[...] = a.reshape(1, 1)


def _blend_call(rw2d, z_e, zq_pure):
    n, d = z_e.shape
    bn = 2048
    return pl.pallas_call(
        _blend_body,
        grid=(n // bn,),
        in_specs=[
            pl.BlockSpec((1, 1), lambda i: (0, 0)),
            pl.BlockSpec((bn, d), lambda i: (i, 0)),
            pl.BlockSpec((bn, d), lambda i: (i, 0)),
        ],
        out_specs=[
            pl.BlockSpec((bn, d), lambda i: (i, 0)),
            pl.BlockSpec((1, 1), lambda i: (0, 0)),
        ],
        out_shape=[
            jax.ShapeDtypeStruct((n, d), jnp.float32),
            jax.ShapeDtypeStruct((1, 1), jnp.float32),
        ],
        compiler_params=pltpu.CompilerParams(
            dimension_semantics=("arbitrary",)),
    )(rw2d, z_e, zq_pure)


def kernel(z_e, embeddings, logit_scale, residual_weight):
    n, d = z_e.shape
    scale2d = jnp.reshape(logit_scale, (1, 1)).astype(jnp.float32)
    rw2d = jnp.reshape(residual_weight, (1, 1)).astype(jnp.float32)

    idx_col, ppl = _stats_call(z_e, embeddings, scale2d)
    indices = jnp.reshape(idx_col, (n,))

    zq_pure = _gather_call(embeddings, jnp.reshape(indices, (-1, 128)), n, d)
    z_q, alpha2d = _blend_call(rw2d, z_e, zq_pure)

    perplexity = jnp.reshape(ppl, ())
    alpha = jnp.reshape(alpha2d, ())
    commitment_loss = jnp.zeros((), jnp.float32)
    return (z_q, indices, perplexity, alpha, commitment_loss)
